# Initial kernel scaffold; baseline (speedup 1.0000x reference)
import jax
import jax.numpy as jnp
from jax.experimental import pallas as pl


def kernel(node_states, context_state, node_graph_ids, W, b):
    raise NotImplementedError("write your pallas kernel here")


# speedup vs baseline: 5.6813x; 5.6813x over previous
import jax
import jax.numpy as jnp
from jax import lax
from jax.experimental import pallas as pl
from jax.experimental.pallas import tpu as pltpu

N_NODES = 100000
NUM_GRAPHS = 512
D_FEAT = 128
BLK = 1000
GRID = N_NODES // BLK


def _tc_body(ids_ref, x_ref, ctx_ref, w_ref, b_ref, out_ref, acc_ref, cnt_ref):
    g = pl.program_id(0)

    ids = ids_ref[0]
    seg = lax.broadcasted_iota(jnp.int32, (NUM_GRAPHS, BLK), 0)
    onehot_t = (seg == ids).astype(jnp.float32)

    partial = lax.dot_general(
        onehot_t, x_ref[...],
        (((1,), (0,)), ((), ())),
        preferred_element_type=jnp.float32,
    )
    cnt_partial = jnp.sum(onehot_t, axis=1, keepdims=True)

    @pl.when(g == 0)
    def _init():
        acc_ref[...] = partial
        cnt_ref[...] = cnt_partial

    @pl.when(g > 0)
    def _acc():
        acc_ref[...] += partial
        cnt_ref[...] += cnt_partial

    @pl.when(g == GRID - 1)
    def _finish():
        cnt = jnp.maximum(cnt_ref[...], 1.0)
        pooled = acc_ref[...] * (1.0 / cnt)
        w_top = w_ref[0:D_FEAT, :]
        w_bot = w_ref[D_FEAT:2 * D_FEAT, :]
        z = lax.dot_general(ctx_ref[...], w_top, (((1,), (0,)), ((), ())),
                            preferred_element_type=jnp.float32)
        z += lax.dot_general(pooled, w_bot, (((1,), (0,)), ((), ())),
                             preferred_element_type=jnp.float32)
        out_ref[...] = jnp.maximum(z + b_ref[...], 0.0)


def kernel(node_states, context_state, node_graph_ids, W, b):
    ids3 = node_graph_ids.astype(jnp.int32).reshape(GRID, 1, BLK)
    b2 = b.reshape(1, D_FEAT)

    out = pl.pallas_call(
        _tc_body,
        grid=(GRID,),
        in_specs=[
            pl.BlockSpec((1, 1, BLK), lambda g: (g, 0, 0)),
            pl.BlockSpec((BLK, D_FEAT), lambda g: (g, 0)),
            pl.BlockSpec((NUM_GRAPHS, D_FEAT), lambda g: (0, 0)),
            pl.BlockSpec((2 * D_FEAT, D_FEAT), lambda g: (0, 0)),
            pl.BlockSpec((1, D_FEAT), lambda g: (0, 0)),
        ],
        out_specs=pl.BlockSpec((NUM_GRAPHS, D_FEAT), lambda g: (0, 0)),
        out_shape=jax.ShapeDtypeStruct((NUM_GRAPHS, D_FEAT), jnp.float32),
        scratch_shapes=[
            pltpu.VMEM((NUM_GRAPHS, D_FEAT), jnp.float32),
            pltpu.VMEM((NUM_GRAPHS, 1), jnp.float32),
        ],
    )(ids3, node_states, context_state, W, b2)
    return out
